# Initial kernel scaffold; baseline (speedup 1.0000x reference)
import jax
import jax.numpy as jnp
from jax.experimental import pallas as pl


def kernel(pred_logits, pred_boxes, orig_target_sizes):
    raise NotImplementedError("write your pallas kernel here")


# speedup vs baseline: 1.2960x; 1.2960x over previous
import jax
import jax.numpy as jnp
from jax.experimental import pallas as pl
from jax.experimental.pallas import tpu as pltpu

_C = 80
_K = 300
_KP = 304
_QR, _QC = 40, 125
_N = 5000
_BIG = 2**30


def _body(lq_ref, lm_ref, boxes_ref, sizes_ref,
          labels_ref, boxes_out_ref, scores_ref,
          m_ref, pmask_ref, qscr_ref):
    m_ref[...] = jnp.max(1.0 / (1.0 + jnp.exp(-lm_ref[0])), axis=-1)
    pmask_ref[...] = jnp.zeros_like(pmask_ref)
    riota = jax.lax.broadcasted_iota(jnp.int32, (_QR, _QC), 0)
    ciota = jax.lax.broadcasted_iota(jnp.int32, (_QR, _QC), 1)
    qiota = riota * _QC + ciota
    cls_iota = jax.lax.broadcasted_iota(jnp.int32, (1, _C), 1)
    neg_inf = jnp.float32(-jnp.inf)

    def step(j, carry):
        m = m_ref[...]
        v = jnp.max(m)
        q = jnp.min(jnp.where(m == v, qiota, _BIG))
        row = 1.0 / (1.0 + jnp.exp(-lq_ref[0, pl.ds(q, 1), :]))
        prow = pmask_ref[pl.ds(q, 1), :]
        eff = row + prow
        cls = jnp.min(jnp.where(eff == v, cls_iota, _BIG))
        hit = cls_iota == cls
        pmask_ref[pl.ds(q, 1), :] = jnp.where(hit, neg_inf, prow)
        nm = jnp.max(jnp.where(hit, neg_inf, eff))
        m_ref[...] = jnp.where(qiota == q, nm, m)
        labels_ref[0, pl.ds(j, 1), :] = jnp.full((1, 1), cls, jnp.int32)
        scores_ref[0, pl.ds(j, 1), :] = jnp.full((1, 1), v, jnp.float32)
        qscr_ref[pl.ds(j, 1), :] = jnp.full((1, 1), q, jnp.int32)
        return carry

    jax.lax.fori_loop(0, _K, step, 0)

    qv = qscr_ref[...]
    qoh = (qv == jax.lax.broadcasted_iota(jnp.int32, (_KP, _N), 1)
           ).astype(jnp.float32)
    gathered = jnp.dot(qoh, boxes_ref[0],
                       preferred_element_type=jnp.float32)
    s = sizes_ref[0].astype(jnp.float32)
    scale = jnp.concatenate([s, s], axis=1)
    cxcy = gathered[:, 0:2]
    wh = gathered[:, 2:4]
    xyxy = jnp.concatenate([cxcy - wh * 0.5, cxcy + wh * 0.5], axis=1)
    boxes_out_ref[0, :, :] = (xyxy * scale)[:_K]


def kernel(pred_logits, pred_boxes, orig_target_sizes):
    B = pred_logits.shape[0]
    lm = pred_logits.reshape(B, _QR, _QC, _C)
    sizes3 = orig_target_sizes.reshape(B, 1, 2)
    labels, boxes, scores = pl.pallas_call(
        _body,
        grid=(B,),
        in_specs=[
            pl.BlockSpec((1, _N, _C), lambda b: (b, 0, 0)),
            pl.BlockSpec((1, _QR, _QC, _C), lambda b: (b, 0, 0, 0)),
            pl.BlockSpec((1, _N, 4), lambda b: (b, 0, 0)),
            pl.BlockSpec((1, 1, 2), lambda b: (b, 0, 0)),
        ],
        out_specs=[
            pl.BlockSpec((1, _K, 1), lambda b: (b, 0, 0)),
            pl.BlockSpec((1, _K, 4), lambda b: (b, 0, 0)),
            pl.BlockSpec((1, _K, 1), lambda b: (b, 0, 0)),
        ],
        out_shape=[
            jax.ShapeDtypeStruct((B, _K, 1), jnp.int32),
            jax.ShapeDtypeStruct((B, _K, 4), jnp.float32),
            jax.ShapeDtypeStruct((B, _K, 1), jnp.float32),
        ],
        scratch_shapes=[
            pltpu.VMEM((_QR, _QC), jnp.float32),
            pltpu.VMEM((_N, _C), jnp.float32),
            pltpu.VMEM((_KP, 1), jnp.int32),
        ],
    )(pred_logits, lm, pred_boxes, sizes3)
    return labels[..., 0], boxes, scores[..., 0]


# speedup vs baseline: 1.5924x; 1.2287x over previous
import jax
import jax.numpy as jnp
from jax.experimental import pallas as pl
from jax.experimental.pallas import tpu as pltpu

_B = 16
_C = 80
_K = 300
_KP = 304
_QR, _QC = 40, 125
_N = 5000
_BIG = 2**30


def _sig(x):
    return 1.0 / (1.0 + jnp.exp(-x))


def _select_body(lm_ref, labels_ref, scores_ref, qout_ref, s_ref, m_ref):
    g = pl.program_id(0)
    nb = s_ref.shape[0]

    @pl.when(g < nb)
    def _fill():
        x = _sig(lm_ref[...])
        s_ref[pl.ds(g, 1)] = x
        m_ref[pl.ds(g, 1)] = jnp.max(x, axis=-1)

    @pl.when(g == nb)
    def _extract():
        qout_ref[...] = jnp.zeros_like(qout_ref)
        riota = jax.lax.broadcasted_iota(jnp.int32, (_QR, _QC), 0)
        ciota = jax.lax.broadcasted_iota(jnp.int32, (_QR, _QC), 1)
        qiota = riota * _QC + ciota
        cls_iota = jax.lax.broadcasted_iota(jnp.int32, (1, _C), 1)
        neg_inf = jnp.float32(-jnp.inf)

        def step(j, carry):
            for b in range(nb):
                m = m_ref[b]
                v = jnp.max(m)
                q = jnp.min(jnp.where(m == v, qiota, _BIG))
                qr = q // _QC
                qc = q - qr * _QC
                eff = s_ref[b, pl.ds(qr, 1), pl.ds(qc, 1), :][0]
                cls = jnp.min(jnp.where(eff == v, cls_iota, _BIG))
                hit = cls_iota == cls
                masked = jnp.where(hit, neg_inf, eff)
                s_ref[b, pl.ds(qr, 1), pl.ds(qc, 1), :] = masked[None]
                nm = jnp.max(masked)
                m_ref[b] = jnp.where(qiota == q, nm, m)
                labels_ref[b, pl.ds(j, 1), :] = jnp.full((1, 1), cls, jnp.int32)
                scores_ref[b, pl.ds(j, 1), :] = jnp.full((1, 1), v, jnp.float32)
                qout_ref[b, pl.ds(j, 1), :] = jnp.full((1, 1), q, jnp.int32)
            return carry

        jax.lax.fori_loop(0, _K, step, 0)


def _boxes_body(boxes_ref, qin_ref, sizes_ref, boxes_out_ref):
    qv = qin_ref[0]
    qoh = (qv == jax.lax.broadcasted_iota(jnp.int32, (_KP, _N), 1)
           ).astype(jnp.float32)
    gathered = jnp.dot(qoh, boxes_ref[0],
                       preferred_element_type=jnp.float32)
    s = sizes_ref[0].astype(jnp.float32)
    scale = jnp.concatenate([s, s], axis=1)
    cxcy = gathered[:, 0:2]
    wh = gathered[:, 2:4]
    xyxy = jnp.concatenate([cxcy - wh * 0.5, cxcy + wh * 0.5], axis=1)
    boxes_out_ref[0, :, :] = (xyxy * scale)[:_K]


def kernel(pred_logits, pred_boxes, orig_target_sizes):
    B = pred_logits.shape[0]
    lm = pred_logits.reshape(B, _QR, _QC, _C)
    sizes3 = orig_target_sizes.reshape(B, 1, 2)
    labels, scores, qout = pl.pallas_call(
        _select_body,
        grid=(B + 1,),
        in_specs=[
            pl.BlockSpec((1, _QR, _QC, _C),
                         lambda g: (jnp.minimum(g, B - 1), 0, 0, 0)),
        ],
        out_specs=[
            pl.BlockSpec((B, _K, 1), lambda g: (0, 0, 0)),
            pl.BlockSpec((B, _K, 1), lambda g: (0, 0, 0)),
            pl.BlockSpec((B, _KP, 1), lambda g: (0, 0, 0)),
        ],
        out_shape=[
            jax.ShapeDtypeStruct((B, _K, 1), jnp.int32),
            jax.ShapeDtypeStruct((B, _K, 1), jnp.float32),
            jax.ShapeDtypeStruct((B, _KP, 1), jnp.int32),
        ],
        scratch_shapes=[
            pltpu.VMEM((B, _QR, _QC, _C), jnp.float32),
            pltpu.VMEM((B, _QR, _QC), jnp.float32),
        ],
    )(lm)
    boxes = pl.pallas_call(
        _boxes_body,
        grid=(B,),
        in_specs=[
            pl.BlockSpec((1, _N, 4), lambda b: (b, 0, 0)),
            pl.BlockSpec((1, _KP, 1), lambda b: (b, 0, 0)),
            pl.BlockSpec((1, 1, 2), lambda b: (b, 0, 0)),
        ],
        out_specs=pl.BlockSpec((1, _K, 4), lambda b: (b, 0, 0)),
        out_shape=jax.ShapeDtypeStruct((B, _K, 4), jnp.float32),
    )(pred_boxes, qout, sizes3)
    return labels[..., 0], boxes, scores[..., 0]


# speedup vs baseline: 2.4001x; 1.5072x over previous
import jax
import jax.numpy as jnp
from jax.experimental import pallas as pl
from jax.experimental.pallas import tpu as pltpu

_B = 16
_C = 80
_K = 300
_KP = 304
_QR, _QC = 40, 125
_N = 5000
_BIG = 2**30


def _sig(x):
    return 1.0 / (1.0 + jnp.exp(-x))


def _select_body(lm_ref, labels_ref, scores_ref, qout_ref, s_ref, m_ref):
    g = pl.program_id(0)
    nb = s_ref.shape[0]

    @pl.when(g < nb)
    def _fill():
        x = _sig(lm_ref[...])
        s_ref[pl.ds(g, 1)] = x
        m_ref[pl.ds(g, 1)] = jnp.max(x, axis=-1)

    @pl.when(g == nb)
    def _extract():
        qout_ref[...] = jnp.zeros_like(qout_ref)
        riota = jax.lax.broadcasted_iota(jnp.int32, (_QR, _QC), 0)
        ciota = jax.lax.broadcasted_iota(jnp.int32, (_QR, _QC), 1)
        qiota = riota * _QC + ciota
        cls_iota = jax.lax.broadcasted_iota(jnp.int32, (1, _C), 1)
        neg_inf = jnp.float32(-jnp.inf)

        qiota3 = qiota[None]

        def step(j, carry):
            m_all = m_ref[...]
            v_all = jnp.max(m_all, axis=(1, 2), keepdims=True)
            q_all = jnp.min(jnp.where(m_all == v_all, qiota3, _BIG),
                            axis=(1, 2), keepdims=True)
            for b in range(nb):
                m = m_all[b]
                v = jnp.max(v_all[b])
                q = jnp.min(q_all[b])
                qr = q // _QC
                qc = q - qr * _QC
                eff = s_ref[b, pl.ds(qr, 1), pl.ds(qc, 1), :][0]
                cls = jnp.min(jnp.where(eff == v, cls_iota, _BIG))
                hit = cls_iota == cls
                masked = jnp.where(hit, neg_inf, eff)
                s_ref[b, pl.ds(qr, 1), pl.ds(qc, 1), :] = masked[None]
                nm = jnp.max(masked)
                m_ref[b] = jnp.where(qiota == q, nm, m)
                labels_ref[b, pl.ds(j, 1), :] = jnp.full((1, 1), cls, jnp.int32)
                scores_ref[b, pl.ds(j, 1), :] = jnp.full((1, 1), v, jnp.float32)
                qout_ref[b, pl.ds(j, 1), :] = jnp.full((1, 1), q, jnp.int32)
            return carry

        jax.lax.fori_loop(0, _K, step, 0)


def _boxes_body(boxes_ref, qin_ref, sizes_ref, boxes_out_ref):
    qv = qin_ref[0]
    qoh = (qv == jax.lax.broadcasted_iota(jnp.int32, (_KP, _N), 1)
           ).astype(jnp.float32)
    gathered = jnp.dot(qoh, boxes_ref[0],
                       preferred_element_type=jnp.float32)
    s = sizes_ref[0].astype(jnp.float32)
    scale = jnp.concatenate([s, s], axis=1)
    cxcy = gathered[:, 0:2]
    wh = gathered[:, 2:4]
    xyxy = jnp.concatenate([cxcy - wh * 0.5, cxcy + wh * 0.5], axis=1)
    boxes_out_ref[0, :, :] = (xyxy * scale)[:_K]


def kernel(pred_logits, pred_boxes, orig_target_sizes):
    B = pred_logits.shape[0]
    lm = pred_logits.reshape(B, _QR, _QC, _C)
    sizes3 = orig_target_sizes.reshape(B, 1, 2)
    labels, scores, qout = pl.pallas_call(
        _select_body,
        grid=(B + 1,),
        in_specs=[
            pl.BlockSpec((1, _QR, _QC, _C),
                         lambda g: (jnp.minimum(g, B - 1), 0, 0, 0)),
        ],
        out_specs=[
            pl.BlockSpec((B, _K, 1), lambda g: (0, 0, 0)),
            pl.BlockSpec((B, _K, 1), lambda g: (0, 0, 0)),
            pl.BlockSpec((B, _KP, 1), lambda g: (0, 0, 0)),
        ],
        out_shape=[
            jax.ShapeDtypeStruct((B, _K, 1), jnp.int32),
            jax.ShapeDtypeStruct((B, _K, 1), jnp.float32),
            jax.ShapeDtypeStruct((B, _KP, 1), jnp.int32),
        ],
        scratch_shapes=[
            pltpu.VMEM((B, _QR, _QC, _C), jnp.float32),
            pltpu.VMEM((B, _QR, _QC), jnp.float32),
        ],
    )(lm)
    boxes = pl.pallas_call(
        _boxes_body,
        grid=(B,),
        in_specs=[
            pl.BlockSpec((1, _N, 4), lambda b: (b, 0, 0)),
            pl.BlockSpec((1, _KP, 1), lambda b: (b, 0, 0)),
            pl.BlockSpec((1, 1, 2), lambda b: (b, 0, 0)),
        ],
        out_specs=pl.BlockSpec((1, _K, 4), lambda b: (b, 0, 0)),
        out_shape=jax.ShapeDtypeStruct((B, _K, 4), jnp.float32),
    )(pred_boxes, qout, sizes3)
    return labels[..., 0], boxes, scores[..., 0]


# speedup vs baseline: 2.4084x; 1.0034x over previous
import jax
import jax.numpy as jnp
from jax.experimental import pallas as pl
from jax.experimental.pallas import tpu as pltpu

_B = 16
_C = 80
_K = 300
_KP = 304
_QR, _QC = 40, 125
_N = 5000
_BIG = 2**30


def _sig(x):
    return 1.0 / (1.0 + jnp.exp(-x))


def _select_body(lm_ref, lq_ref, fidx_ref, scores_ref, s_ref, m_ref):
    g = pl.program_id(0)
    nb = s_ref.shape[0]

    @pl.when(g < nb)
    def _fill():
        s_ref[pl.ds(g, 1)] = _sig(lq_ref[...])
        m_ref[pl.ds(g, 1)] = jnp.max(_sig(lm_ref[...]), axis=-1)

    @pl.when(g == nb)
    def _extract():
        fidx_ref[...] = jnp.zeros_like(fidx_ref)
        riota = jax.lax.broadcasted_iota(jnp.int32, (_QR, _QC), 0)
        ciota = jax.lax.broadcasted_iota(jnp.int32, (_QR, _QC), 1)
        qiota = riota * _QC + ciota
        cls_iota = jax.lax.broadcasted_iota(jnp.int32, (1, _C), 1)
        neg_inf = jnp.float32(-jnp.inf)

        qiota3 = qiota[None]

        def step(j, carry):
            m_all = m_ref[...]
            v_all = jnp.max(m_all, axis=(1, 2), keepdims=True)
            q_all = jnp.min(jnp.where(m_all == v_all, qiota3, _BIG),
                            axis=(1, 2), keepdims=True)
            for b in range(nb):
                m = m_all[b]
                v = jnp.max(v_all[b])
                q = jnp.min(q_all[b])
                eff = s_ref[b, pl.ds(q, 1), :]
                cls = jnp.min(jnp.where(eff == v, cls_iota, _BIG))
                hit = cls_iota == cls
                masked = jnp.where(hit, neg_inf, eff)
                s_ref[b, pl.ds(q, 1), :] = masked
                nm = jnp.max(masked)
                m_ref[b] = jnp.where(qiota == q, nm, m)
                fidx_ref[b, pl.ds(j, 1), :] = jnp.full((1, 1), q * _C + cls,
                                                       jnp.int32)
                scores_ref[b, pl.ds(j, 1), :] = jnp.full((1, 1), v, jnp.float32)
            return carry

        jax.lax.fori_loop(0, _K, step, 0)


def _boxes_body(boxes_ref, fidx_ref, sizes_ref, boxes_out_ref, labels_ref):
    fv = fidx_ref[0]
    qv = fv // _C
    labels_ref[0, :, :] = (fv - qv * _C)[:_K]
    qoh = (qv == jax.lax.broadcasted_iota(jnp.int32, (_KP, _N), 1)
           ).astype(jnp.float32)
    gathered = jnp.dot(qoh, boxes_ref[0],
                       preferred_element_type=jnp.float32)
    s = sizes_ref[0].astype(jnp.float32)
    scale = jnp.concatenate([s, s], axis=1)
    cxcy = gathered[:, 0:2]
    wh = gathered[:, 2:4]
    xyxy = jnp.concatenate([cxcy - wh * 0.5, cxcy + wh * 0.5], axis=1)
    boxes_out_ref[0, :, :] = (xyxy * scale)[:_K]


def kernel(pred_logits, pred_boxes, orig_target_sizes):
    B = pred_logits.shape[0]
    lm = pred_logits.reshape(B, _QR, _QC, _C)
    sizes3 = orig_target_sizes.reshape(B, 1, 2)
    fidx, scores = pl.pallas_call(
        _select_body,
        grid=(B + 1,),
        in_specs=[
            pl.BlockSpec((1, _QR, _QC, _C),
                         lambda g: (jnp.minimum(g, B - 1), 0, 0, 0)),
            pl.BlockSpec((1, _N, _C),
                         lambda g: (jnp.minimum(g, B - 1), 0, 0)),
        ],
        out_specs=[
            pl.BlockSpec((B, _KP, 1), lambda g: (0, 0, 0)),
            pl.BlockSpec((B, _K, 1), lambda g: (0, 0, 0)),
        ],
        out_shape=[
            jax.ShapeDtypeStruct((B, _KP, 1), jnp.int32),
            jax.ShapeDtypeStruct((B, _K, 1), jnp.float32),
        ],
        scratch_shapes=[
            pltpu.VMEM((B, _N, _C), jnp.float32),
            pltpu.VMEM((B, _QR, _QC), jnp.float32),
        ],
    )(lm, pred_logits)
    boxes, labels = pl.pallas_call(
        _boxes_body,
        grid=(B,),
        in_specs=[
            pl.BlockSpec((1, _N, 4), lambda b: (b, 0, 0)),
            pl.BlockSpec((1, _KP, 1), lambda b: (b, 0, 0)),
            pl.BlockSpec((1, 1, 2), lambda b: (b, 0, 0)),
        ],
        out_specs=[
            pl.BlockSpec((1, _K, 4), lambda b: (b, 0, 0)),
            pl.BlockSpec((1, _K, 1), lambda b: (b, 0, 0)),
        ],
        out_shape=[
            jax.ShapeDtypeStruct((B, _K, 4), jnp.float32),
            jax.ShapeDtypeStruct((B, _K, 1), jnp.int32),
        ],
    )(pred_boxes, fidx, sizes3)
    return labels[..., 0], boxes, scores[..., 0]
